# baseline (speedup 1.0000x reference)
import jax
import jax.numpy as jnp
from jax.experimental import pallas as pl
from jax.experimental.pallas import tpu as pltpu

_ROWS = 8192
_COLS = 1024
_BLOCK_ROWS = 512
_GRID = _ROWS // _BLOCK_ROWS


def _l1_body(pred_ref, gt_ref, out_ref, acc_ref):
    i = pl.program_id(0)

    @pl.when(i == 0)
    def _init():
        acc_ref[0] = 0.0
        acc_ref[1] = 0.0

    p = pred_ref[...]
    g = gt_ref[0]
    m = gt_ref[1] > 0.0
    diff = jnp.where(m, jnp.abs(p - g), 0.0)
    acc_ref[0] += jnp.sum(diff)
    acc_ref[1] += jnp.sum(m.astype(jnp.float32))

    @pl.when(i == _GRID - 1)
    def _fin():
        out_ref[0, 0] = acc_ref[0] / acc_ref[1]


def kernel(pred, gt):
    pred2 = pred.reshape(_ROWS, _COLS)
    gt2 = gt.reshape(2, _ROWS, _COLS)
    out = pl.pallas_call(
        _l1_body,
        grid=(_GRID,),
        in_specs=[
            pl.BlockSpec((_BLOCK_ROWS, _COLS), lambda i: (i, 0)),
            pl.BlockSpec((2, _BLOCK_ROWS, _COLS), lambda i: (0, i, 0)),
        ],
        out_specs=pl.BlockSpec(memory_space=pltpu.SMEM),
        out_shape=jax.ShapeDtypeStruct((1, 1), jnp.float32),
        scratch_shapes=[pltpu.SMEM((2,), jnp.float32)],
    )(pred2, gt2)
    return out[0, 0]
